# baseline (device time: 74914 ns/iter reference)
import jax
import jax.numpy as jnp
from jax import lax
from jax.experimental import pallas as pl
from jax.experimental.pallas import tpu as pltpu

N_DEV = 8

_ROT_MASKS = ((1, 3, 4), (3, 4, 1), (4, 1, 3))
_ROT_AS = ((3, 1, 4), (2, 4, 1), (4, 3, 1))
_GEN_UNITS = (
    (3, 3, 3),
    (3, 3, 3),
    (3, 3, 2),
    (2, 2, 2),
)


def _build_parts():
    parts = []
    gens = []
    off = 0
    for gu in _GEN_UNITS:
        gen = []
        for r, u in enumerate(gu):
            gen.append(len(parts))
            parts.append((u * 128, off, _ROT_MASKS[r], _ROT_AS[r]))
            off += u * 128
        gens.append(tuple(gen))
    return tuple(parts), tuple(gens)


_PARTS, _GENS = _build_parts()
_NP = len(_PARTS)

_BF16 = jnp.bfloat16
_F32 = jnp.float32


def _parity(v):
    return (v ^ (v >> 1) ^ (v >> 2)) & 1


def kernel(t):
    m_per, n = t.shape
    assert m_per == _PARTS[-1][1] + _PARTS[-1][0]

    def body(x_hbm, out_hbm, *scratch):
        xvs = scratch[0 * _NP:1 * _NP]
        ovs = scratch[1 * _NP:2 * _NP]
        s1s = scratch[2 * _NP:3 * _NP]
        r1s = scratch[3 * _NP:4 * _NP]
        accs = scratch[4 * _NP:5 * _NP]
        s2s = scratch[5 * _NP:6 * _NP]
        r2s = scratch[6 * _NP:7 * _NP]
        s3s = scratch[7 * _NP:8 * _NP]
        r3s = scratch[8 * _NP:9 * _NP]
        gs = scratch[9 * _NP:10 * _NP]
        g2s = scratch[10 * _NP:11 * _NP]
        send_sems = scratch[11 * _NP]
        recv_sems = scratch[11 * _NP + 1]
        xsems = scratch[11 * _NP + 2]
        osems = scratch[11 * _NP + 3]

        d = lax.axis_index("i")

        xcopies = []
        for p, (L, off, _, _) in enumerate(_PARTS):
            cp = pltpu.make_async_copy(
                x_hbm.at[pl.ds(off, L), :], xvs[p], xsems.at[p]
            )
            cp.start()
            xcopies.append(cp)

        barrier = pltpu.get_barrier_semaphore()
        for m in (1, 3, 4):
            pl.semaphore_signal(
                barrier, inc=1, device_id=(d ^ m,),
                device_id_type=pl.DeviceIdType.MESH,
            )
        pl.semaphore_wait(barrier, 3)

        fs = [
            (_parity(d & a1), _parity(d & a2), _parity(d & a3))
            for _, _, _, (a1, a2, a3) in _PARTS
        ]

        def exch(p, step, src, dst, mask):
            rdma = pltpu.make_async_remote_copy(
                src_ref=src,
                dst_ref=dst,
                send_sem=send_sems.at[p, step],
                recv_sem=recv_sems.at[p, step],
                device_id=(d ^ mask,),
                device_id_type=pl.DeviceIdType.MESH,
            )
            rdma.start()
            return rdma

        rs1 = [None] * _NP
        rs2 = [None] * _NP
        rs3 = [None] * _NP
        ag1 = [None] * _NP
        ag2 = [None] * _NP
        ag3 = [None] * _NP
        outcopies = [[None] * 4 for _ in range(_NP)]

        def put(p, k, vsrc, goff, rows):
            cp = pltpu.make_async_copy(
                vsrc, out_hbm.at[pl.ds(goff, rows), :], osems.at[p, k]
            )
            cp.start()
            outcopies[p][k] = cp

        def phase_rs1(group):
            for p in group:
                L, off, (m1, _, _), _ = _PARTS[p]
                xcopies[p].wait()
                f1 = fs[p][0]
                send_off = (1 - f1) * (L // 2)
                s1s[p][...] = xvs[p][pl.ds(send_off, L // 2), :].astype(_BF16)
                rs1[p] = exch(p, 0, s1s[p], r1s[p], m1)

        def phase_rs2(group):
            for p in group:
                L, _, (_, m2, _), _ = _PARTS[p]
                rs1[p].wait()
                f1, f2, _ = fs[p]
                my_off = f1 * (L // 2)
                send_q = (1 - f2) * (L // 4)
                s2s[p][...] = (
                    xvs[p][pl.ds(my_off + send_q, L // 4), :]
                    + r1s[p][pl.ds(send_q, L // 4), :]
                ).astype(_BF16)
                rs2[p] = exch(p, 1, s2s[p], r2s[p], m2)
            for p in group:
                L, _, _, _ = _PARTS[p]
                f1, f2, _ = fs[p]
                my_off = f1 * (L // 2)
                keep_q = f2 * (L // 4)
                accs[p][...] = (
                    xvs[p][pl.ds(my_off + keep_q, L // 4), :]
                    + r1s[p][pl.ds(keep_q, L // 4), :]
                )

        def phase_rs3(group):
            for p in group:
                L, _, (_, _, m3), _ = _PARTS[p]
                rs2[p].wait()
                _, _, f3 = fs[p]
                send_e = (1 - f3) * (L // 8)
                s3s[p][...] = (
                    accs[p][pl.ds(send_e, L // 8), :]
                    + r2s[p][pl.ds(send_e, L // 8), :]
                ).astype(_BF16)
                rs3[p] = exch(p, 2, s3s[p], r3s[p], m3)
            for p in group:
                L, _, _, _ = _PARTS[p]
                _, _, f3 = fs[p]
                keep_e = f3 * (L // 8)
                accs[p][pl.ds(0, L // 8), :] = (
                    accs[p][pl.ds(keep_e, L // 8), :]
                    + r2s[p][pl.ds(keep_e, L // 8), :]
                )

        def phase_fin(group):
            for p in group:
                L, off, (_, _, m3), _ = _PARTS[p]
                rs3[p].wait()
                f1, f2, f3 = fs[p]
                s = accs[p][pl.ds(0, L // 8), :] + r3s[p][...]
                loc3 = f1 * (L // 2) + f2 * (L // 4) + f3 * (L // 8)
                r = jnp.maximum(s, 0.0)
                fval = jnp.tanh(s) * s * s + r * r * r
                ovs[p][pl.ds(loc3, L // 8), :] = fval
                put(p, 0, ovs[p].at[pl.ds(loc3, L // 8), :], off + loc3,
                    L // 8)
                gloc3 = f2 * (L // 4) + f3 * (L // 8)
                gs[p][pl.ds(gloc3, L // 8), :] = fval.astype(_BF16)
                blk = gs[p].at[pl.ds(gloc3, L // 8), :]
                ag1[p] = exch(p, 3, blk, blk, m3)

        def phase_ag2(group):
            for p in group:
                L, _, (_, m2, _), _ = _PARTS[p]
                ag1[p].wait()
                f2 = fs[p][1]
                blk = gs[p].at[pl.ds(f2 * (L // 4), L // 4), :]
                ag2[p] = exch(p, 4, blk, blk, m2)
            for p in group:
                L, off, _, _ = _PARTS[p]
                f1, f2, f3 = fs[p]
                gloc = f2 * (L // 4) + (1 - f3) * (L // 8)
                loc = f1 * (L // 2) + gloc
                ovs[p][pl.ds(loc, L // 8), :] = gs[p][
                    pl.ds(gloc, L // 8), :
                ].astype(_F32)
                put(p, 1, ovs[p].at[pl.ds(loc, L // 8), :], off + loc,
                    L // 8)

        def phase_ag3(group):
            for p in group:
                L, _, (m1, _, _), _ = _PARTS[p]
                ag2[p].wait()
                ag3[p] = exch(p, 5, gs[p], g2s[p], m1)
            for p in group:
                L, off, _, _ = _PARTS[p]
                f1, f2, _ = fs[p]
                gloc = (1 - f2) * (L // 4)
                loc = f1 * (L // 2) + gloc
                ovs[p][pl.ds(loc, L // 4), :] = gs[p][
                    pl.ds(gloc, L // 4), :
                ].astype(_F32)
                put(p, 2, ovs[p].at[pl.ds(loc, L // 4), :], off + loc,
                    L // 4)

        def phase_tail(group):
            for p in group:
                L, off, _, _ = _PARTS[p]
                ag3[p].wait()
                f1 = fs[p][0]
                loc = (1 - f1) * (L // 2)
                ovs[p][pl.ds(loc, L // 2), :] = g2s[p][...].astype(_F32)
                put(p, 3, ovs[p].at[pl.ds(loc, L // 2), :], off + loc,
                    L // 2)

        for phase in (phase_rs1, phase_rs2, phase_rs3, phase_fin,
                      phase_ag2, phase_ag3, phase_tail):
            for gen in _GENS:
                phase(gen)

        for p in range(_NP):
            for k in range(4):
                outcopies[p][k].wait()

    scratch_shapes = []
    for L, _, _, _ in _PARTS:
        scratch_shapes.append(pltpu.VMEM((L, 1024), _F32))
    for L, _, _, _ in _PARTS:
        scratch_shapes.append(pltpu.VMEM((L, 1024), _F32))
    for L, _, _, _ in _PARTS:
        scratch_shapes.append(pltpu.VMEM((L // 2, 1024), _BF16))
    for L, _, _, _ in _PARTS:
        scratch_shapes.append(pltpu.VMEM((L // 2, 1024), _BF16))
    for L, _, _, _ in _PARTS:
        scratch_shapes.append(pltpu.VMEM((L // 4, 1024), _F32))
    for L, _, _, _ in _PARTS:
        scratch_shapes.append(pltpu.VMEM((L // 4, 1024), _BF16))
    for L, _, _, _ in _PARTS:
        scratch_shapes.append(pltpu.VMEM((L // 4, 1024), _BF16))
    for L, _, _, _ in _PARTS:
        scratch_shapes.append(pltpu.VMEM((L // 8, 1024), _BF16))
    for L, _, _, _ in _PARTS:
        scratch_shapes.append(pltpu.VMEM((L // 8, 1024), _BF16))
    for L, _, _, _ in _PARTS:
        scratch_shapes.append(pltpu.VMEM((L // 2, 1024), _BF16))
    for L, _, _, _ in _PARTS:
        scratch_shapes.append(pltpu.VMEM((L // 2, 1024), _BF16))
    scratch_shapes.append(pltpu.SemaphoreType.DMA((_NP, 6)))
    scratch_shapes.append(pltpu.SemaphoreType.DMA((_NP, 6)))
    scratch_shapes.append(pltpu.SemaphoreType.DMA((_NP,)))
    scratch_shapes.append(pltpu.SemaphoreType.DMA((_NP, 4)))

    return pl.pallas_call(
        body,
        out_shape=jax.ShapeDtypeStruct((m_per, n), _F32),
        in_specs=[pl.BlockSpec(memory_space=pl.ANY)],
        out_specs=pl.BlockSpec(memory_space=pl.ANY),
        scratch_shapes=scratch_shapes,
        compiler_params=pltpu.CompilerParams(
            collective_id=0, vmem_limit_bytes=100 * 1024 * 1024
        ),
    )(t)
